# baseline (speedup 1.0000x reference)
import functools

import jax
import jax.numpy as jnp
from jax import lax
from jax.experimental import pallas as pl
from jax.experimental.pallas import tpu as pltpu
from jax.experimental.pallas import tpu_sc as plsc

N_NODES = 10000
N_EDGES = 320000
NFEAT = 128
NEMBED = 128
NX = 64
FH = 64

NC = 2
NS = 16
K = 128
G = 8
NB = 2

ROW_BLK = 400
N_ROW_BLKS = N_NODES // ROW_BLK

GROUPS_PER_TILE = -(-N_EDGES // (NS * K * G))
CHUNKS_PER_TILE = GROUPS_PER_TILE * G
E_PAD = CHUNKS_PER_TILE * NS * K
TOTAL_CHUNKS = E_PAD // K
NP = N_NODES + 112
ROWS_PER_TILE_ZERO = NP // NS
ROWS_PER_TILE_CP = 624
CP_TAIL_BASE = NS * ROWS_PER_TILE_CP
CP_TAIL = N_NODES - CP_TAIL_BASE

_mesh = plsc.VectorSubcoreMesh(core_axis_name="c", subcore_axis_name="s")


@functools.partial(
    pl.kernel,
    mesh=_mesh,
    out_type=(jax.ShapeDtypeStruct((N_NODES, FH), jnp.float32),
              jax.ShapeDtypeStruct((N_NODES, FH), jnp.float32)),
    scratch_types=[
        pltpu.VMEM_SHARED((N_NODES, FH), jnp.float32),
        pltpu.VMEM_SHARED((NP, FH), jnp.float32),
        pltpu.VMEM((G, K), jnp.int32),
        pltpu.VMEM((G, K), jnp.int32),
        pltpu.VMEM((NB, K, FH), jnp.float32),
        pltpu.SemaphoreType.DMA,
        pltpu.SemaphoreType.DMA,
    ],
    compiler_params=pltpu.CompilerParams(use_tc_tiling_on_sc=False),
)
def _spmm(h0_hbm, h1_hbm, src_hbm, dst_hbm, zeros_hbm, out0_hbm, out1_hbm,
          hcache, acc, src_v, dst_v, rows_v, sem0, sem1):
    cid = lax.axis_index("c")
    sid = lax.axis_index("s")
    sems = (sem0, sem1)

    z0 = sid * ROWS_PER_TILE_ZERO
    pltpu.sync_copy(zeros_hbm.at[pl.ds(z0, ROWS_PER_TILE_ZERO)],
                    acc.at[pl.ds(z0, ROWS_PER_TILE_ZERO)])
    s0 = sid * ROWS_PER_TILE_CP

    @pl.when(cid == 0)
    def _stage0():
        pltpu.sync_copy(h0_hbm.at[pl.ds(s0, ROWS_PER_TILE_CP)],
                        hcache.at[pl.ds(s0, ROWS_PER_TILE_CP)])

        @pl.when(sid == 0)
        def _tail0():
            pltpu.sync_copy(h0_hbm.at[pl.ds(CP_TAIL_BASE, CP_TAIL)],
                            hcache.at[pl.ds(CP_TAIL_BASE, CP_TAIL)])

    @pl.when(cid == 1)
    def _stage1():
        pltpu.sync_copy(h1_hbm.at[pl.ds(s0, ROWS_PER_TILE_CP)],
                        hcache.at[pl.ds(s0, ROWS_PER_TILE_CP)])

        @pl.when(sid == 0)
        def _tail1():
            pltpu.sync_copy(h1_hbm.at[pl.ds(CP_TAIL_BASE, CP_TAIL)],
                            hcache.at[pl.ds(CP_TAIL_BASE, CP_TAIL)])

    plsc.subcore_barrier()

    tile_chunk0 = sid * CHUNKS_PER_TILE

    def group_body(g, carry):
        row0 = tile_chunk0 + g * G
        pltpu.sync_copy(src_hbm.at[pl.ds(row0, G)], src_v)
        pltpu.sync_copy(dst_hbm.at[pl.ds(row0, G)], dst_v)
        descs = [
            pltpu.async_copy(hcache.at[src_v.at[b]], rows_v.at[b], sems[b])
            for b in range(NB)
        ]
        for j in range(G):
            b = j % NB
            descs[b].wait()
            pltpu.sync_copy(rows_v.at[b], acc.at[dst_v.at[j]], add=True)
            nj = j + NB
            if nj < G:
                descs[b] = pltpu.async_copy(
                    hcache.at[src_v.at[nj]], rows_v.at[b], sems[b])
        return carry

    lax.fori_loop(0, GROUPS_PER_TILE, group_body, 0)
    plsc.subcore_barrier()

    @pl.when(cid == 0)
    def _out0():
        pltpu.sync_copy(acc.at[pl.ds(s0, ROWS_PER_TILE_CP)],
                        out0_hbm.at[pl.ds(s0, ROWS_PER_TILE_CP)])

        @pl.when(sid == 0)
        def _otail0():
            pltpu.sync_copy(acc.at[pl.ds(CP_TAIL_BASE, CP_TAIL)],
                            out0_hbm.at[pl.ds(CP_TAIL_BASE, CP_TAIL)])

    @pl.when(cid == 1)
    def _out1():
        pltpu.sync_copy(acc.at[pl.ds(s0, ROWS_PER_TILE_CP)],
                        out1_hbm.at[pl.ds(s0, ROWS_PER_TILE_CP)])

        @pl.when(sid == 0)
        def _otail1():
            pltpu.sync_copy(acc.at[pl.ds(CP_TAIL_BASE, CP_TAIL)],
                            out1_hbm.at[pl.ds(CP_TAIL_BASE, CP_TAIL)])


def _mm1_body(x_ref, w_ref, out0_ref, out1_ref):
    r = jnp.dot(x_ref[...], w_ref[...], preferred_element_type=jnp.float32)
    out0_ref[...] = r[:, :FH]
    out1_ref[...] = r[:, FH:]


def _mm1(x, w1):
    return pl.pallas_call(
        _mm1_body,
        grid=(N_ROW_BLKS,),
        in_specs=[
            pl.BlockSpec((ROW_BLK, NFEAT), lambda i: (i, 0)),
            pl.BlockSpec((NFEAT, NEMBED), lambda i: (0, 0)),
        ],
        out_specs=[pl.BlockSpec((ROW_BLK, FH), lambda i: (i, 0)),
                   pl.BlockSpec((ROW_BLK, FH), lambda i: (i, 0))],
        out_shape=[jax.ShapeDtypeStruct((N_NODES, FH), jnp.float32),
                   jax.ShapeDtypeStruct((N_NODES, FH), jnp.float32)],
    )(x, w1)


def _relu_body(p0_ref, p1_ref, b1_ref, out0_ref, out1_ref):
    out0_ref[...] = jnp.maximum(p0_ref[...] + b1_ref[0, :FH], 0.0)
    out1_ref[...] = jnp.maximum(p1_ref[...] + b1_ref[0, FH:], 0.0)


def _relu_halves(p0, p1, b1):
    return pl.pallas_call(
        _relu_body,
        grid=(N_ROW_BLKS,),
        in_specs=[
            pl.BlockSpec((ROW_BLK, FH), lambda i: (i, 0)),
            pl.BlockSpec((ROW_BLK, FH), lambda i: (i, 0)),
            pl.BlockSpec((1, NEMBED), lambda i: (0, 0)),
        ],
        out_specs=[pl.BlockSpec((ROW_BLK, FH), lambda i: (i, 0)),
                   pl.BlockSpec((ROW_BLK, FH), lambda i: (i, 0))],
        out_shape=[jax.ShapeDtypeStruct((N_NODES, FH), jnp.float32),
                   jax.ShapeDtypeStruct((N_NODES, FH), jnp.float32)],
    )(p0, p1, b1.reshape(1, NEMBED))


def _lsm_body(q0_ref, q1_ref, w2_ref, b2_ref, out_ref):
    s = jnp.concatenate([q0_ref[...], q1_ref[...]], axis=1)
    a = jnp.dot(s, w2_ref[...], preferred_element_type=jnp.float32) + b2_ref[0, :]
    m = jnp.max(a, axis=1, keepdims=True)
    e = jnp.exp(a - m)
    out_ref[...] = a - m - jnp.log(jnp.sum(e, axis=1, keepdims=True))


def _lsm(q0, q1, w2, b2):
    return pl.pallas_call(
        _lsm_body,
        grid=(N_ROW_BLKS,),
        in_specs=[
            pl.BlockSpec((ROW_BLK, FH), lambda i: (i, 0)),
            pl.BlockSpec((ROW_BLK, FH), lambda i: (i, 0)),
            pl.BlockSpec((NEMBED, NX), lambda i: (0, 0)),
            pl.BlockSpec((1, NX), lambda i: (0, 0)),
        ],
        out_specs=pl.BlockSpec((ROW_BLK, NX), lambda i: (i, 0)),
        out_shape=jax.ShapeDtypeStruct((N_NODES, NX), jnp.float32),
    )(q0, q1, w2, b2.reshape(1, NX))


def kernel(x, edge_index, W1, b1, W2, b2):
    src = edge_index[0].astype(jnp.int32)
    dst = edge_index[1].astype(jnp.int32)
    pad = E_PAD - N_EDGES
    src = jnp.concatenate([src, jnp.zeros((pad,), jnp.int32)])
    dst = jnp.concatenate([dst, jnp.full((pad,), N_NODES, jnp.int32)])
    src = src.reshape(TOTAL_CHUNKS, K)
    dst = dst.reshape(TOTAL_CHUNKS, K)
    zeros = jnp.zeros((NP, FH), jnp.float32)

    h0, h1 = _mm1(x, W1)
    p0, p1 = _spmm(h0, h1, src, dst, zeros)
    r0, r1 = _relu_halves(p0, p1, b1)
    q0, q1 = _spmm(r0, r1, src, dst, zeros)
    return _lsm(q0, q1, W2, b2)


# speedup vs baseline: 1.2581x; 1.2581x over previous
import functools

import jax
import jax.numpy as jnp
from jax import lax
from jax.experimental import pallas as pl
from jax.experimental.pallas import tpu as pltpu
from jax.experimental.pallas import tpu_sc as plsc

N_NODES = 10000
N_EDGES = 320000
NFEAT = 128
NEMBED = 128
NX = 64
FH = 64

NC = 2
NS = 16
K = 128
G = 8
NRB = 4
NG = 2

ROW_BLK = 400
N_ROW_BLKS = N_NODES // ROW_BLK

GROUPS_PER_TILE = -(-N_EDGES // (NS * K * G))
CHUNKS_PER_TILE = GROUPS_PER_TILE * G
E_PAD = CHUNKS_PER_TILE * NS * K
TOTAL_CHUNKS = E_PAD // K
NP = N_NODES + 112
ROWS_PER_TILE_ZERO = NP // NS
ROWS_PER_TILE_CP = 624
CP_TAIL_BASE = NS * ROWS_PER_TILE_CP
CP_TAIL = N_NODES - CP_TAIL_BASE
WR = 78

_mesh = plsc.VectorSubcoreMesh(core_axis_name="c", subcore_axis_name="s")


@functools.partial(
    pl.kernel,
    mesh=_mesh,
    out_type=(jax.ShapeDtypeStruct((N_NODES, FH), jnp.float32),
              jax.ShapeDtypeStruct((N_NODES, FH), jnp.float32)),
    scratch_types=[
        pltpu.VMEM_SHARED((N_NODES, FH), jnp.float32),
        pltpu.VMEM_SHARED((NP, FH), jnp.float32),
        pltpu.VMEM((2, G, K), jnp.int32),
        pltpu.VMEM((2, G, K), jnp.int32),
        pltpu.VMEM((NRB, K, FH), jnp.float32),
        pltpu.VMEM((WR, FH), jnp.float32),
        pltpu.VMEM((1, FH), jnp.float32),
        pltpu.SemaphoreType.DMA,
        pltpu.SemaphoreType.DMA,
        pltpu.SemaphoreType.DMA,
        pltpu.SemaphoreType.DMA,
        pltpu.SemaphoreType.DMA,
        pltpu.SemaphoreType.DMA,
        pltpu.SemaphoreType.DMA,
        pltpu.SemaphoreType.DMA,
        pltpu.SemaphoreType.DMA,
    ],
    compiler_params=pltpu.CompilerParams(use_tc_tiling_on_sc=False),
)
def _gcn_sc(h0_hbm, h1_hbm, src_hbm, dst_hbm, zeros_hbm, b1_hbm,
            out0_hbm, out1_hbm,
            hcache, acc, src_v, dst_v, rows_v, work_v, bias_v,
            isem, g0, g1, g2, g3, s0sem, s1sem, s2sem, s3sem):
    cid = lax.axis_index("c")
    sid = lax.axis_index("s")
    gsems = (g0, g1, g2, g3)
    ssems = (s0sem, s1sem, s2sem, s3sem)

    z0 = sid * ROWS_PER_TILE_ZERO
    s0 = sid * ROWS_PER_TILE_CP
    tile_chunk0 = sid * CHUNKS_PER_TILE

    def zero_acc():
        pltpu.sync_copy(zeros_hbm.at[pl.ds(z0, ROWS_PER_TILE_ZERO)],
                        acc.at[pl.ds(z0, ROWS_PER_TILE_ZERO)])

    zero_acc()
    pltpu.sync_copy(b1_hbm.at[pl.ds(cid, 1)], bias_v)

    @pl.when(cid == 0)
    def _stage0():
        pltpu.sync_copy(h0_hbm.at[pl.ds(s0, ROWS_PER_TILE_CP)],
                        hcache.at[pl.ds(s0, ROWS_PER_TILE_CP)])

        @pl.when(sid == 0)
        def _tail0():
            pltpu.sync_copy(h0_hbm.at[pl.ds(CP_TAIL_BASE, CP_TAIL)],
                            hcache.at[pl.ds(CP_TAIL_BASE, CP_TAIL)])

    @pl.when(cid == 1)
    def _stage1():
        pltpu.sync_copy(h1_hbm.at[pl.ds(s0, ROWS_PER_TILE_CP)],
                        hcache.at[pl.ds(s0, ROWS_PER_TILE_CP)])

        @pl.when(sid == 0)
        def _tail1():
            pltpu.sync_copy(h1_hbm.at[pl.ds(CP_TAIL_BASE, CP_TAIL)],
                            hcache.at[pl.ds(CP_TAIL_BASE, CP_TAIL)])

    plsc.subcore_barrier()

    def edge_pass():
        pltpu.async_copy(src_hbm.at[pl.ds(tile_chunk0, G)], src_v.at[0], isem)
        pltpu.async_copy(dst_hbm.at[pl.ds(tile_chunk0, G)], dst_v.at[0], isem)

        def group_body(g, carry):
            cur = lax.rem(g, 2)
            nxt = 1 - cur
            pltpu.make_async_copy(src_hbm.at[pl.ds(0, G)], src_v.at[0],
                                  isem).wait()
            pltpu.make_async_copy(dst_hbm.at[pl.ds(0, G)], dst_v.at[0],
                                  isem).wait()
            nrow = tile_chunk0 + (g + 1) * G
            pltpu.async_copy(src_hbm.at[pl.ds(nrow, G)], src_v.at[nxt], isem)
            pltpu.async_copy(dst_hbm.at[pl.ds(nrow, G)], dst_v.at[nxt], isem)

            gd = [None] * NRB
            sd = [None] * NRB
            for j in range(NG):
                b = j % NRB
                gd[b] = pltpu.async_copy(
                    hcache.at[src_v.at[cur, j]], rows_v.at[b], gsems[b])
            for j in range(G):
                b = j % NRB
                gd[b].wait()
                sd[b] = pltpu.async_copy(
                    rows_v.at[b], acc.at[dst_v.at[cur, j]], ssems[b],
                    add=True)
                f = j + NG
                if f < G:
                    fb = f % NRB
                    if sd[fb] is not None:
                        sd[fb].wait()
                        sd[fb] = None
                    gd[fb] = pltpu.async_copy(
                        hcache.at[src_v.at[cur, f]], rows_v.at[fb], gsems[fb])
            for b in range(NRB):
                if sd[b] is not None:
                    sd[b].wait()
            return carry

        lax.fori_loop(0, GROUPS_PER_TILE, group_body, 0)
        pltpu.make_async_copy(src_hbm.at[pl.ds(0, G)], src_v.at[0],
                              isem).wait()
        pltpu.make_async_copy(dst_hbm.at[pl.ds(0, G)], dst_v.at[0],
                              isem).wait()

    edge_pass()
    plsc.subcore_barrier()

    def relu_rows(base, nrows):
        pltpu.sync_copy(acc.at[pl.ds(base, nrows)], work_v.at[pl.ds(0, nrows)])

        def row_body(r, carry):
            for c in range(FH // 16):
                sl = pl.ds(c * 16, 16)
                work_v[r, sl] = jnp.maximum(work_v[r, sl] + bias_v[0, sl], 0.0)
            return carry

        lax.fori_loop(0, nrows, row_body, 0)
        pltpu.sync_copy(work_v.at[pl.ds(0, nrows)], hcache.at[pl.ds(base, nrows)])

    for w in range(ROWS_PER_TILE_CP // WR):
        relu_rows(s0 + w * WR, WR)

    @pl.when(sid == 0)
    def _relu_tail():
        relu_rows(CP_TAIL_BASE, CP_TAIL)

    plsc.subcore_barrier()
    zero_acc()
    plsc.subcore_barrier()

    edge_pass()
    plsc.subcore_barrier()

    @pl.when(cid == 0)
    def _out0():
        pltpu.sync_copy(acc.at[pl.ds(s0, ROWS_PER_TILE_CP)],
                        out0_hbm.at[pl.ds(s0, ROWS_PER_TILE_CP)])

        @pl.when(sid == 0)
        def _otail0():
            pltpu.sync_copy(acc.at[pl.ds(CP_TAIL_BASE, CP_TAIL)],
                            out0_hbm.at[pl.ds(CP_TAIL_BASE, CP_TAIL)])

    @pl.when(cid == 1)
    def _out1():
        pltpu.sync_copy(acc.at[pl.ds(s0, ROWS_PER_TILE_CP)],
                        out1_hbm.at[pl.ds(s0, ROWS_PER_TILE_CP)])

        @pl.when(sid == 0)
        def _otail1():
            pltpu.sync_copy(acc.at[pl.ds(CP_TAIL_BASE, CP_TAIL)],
                            out1_hbm.at[pl.ds(CP_TAIL_BASE, CP_TAIL)])


def _mm1_body(x_ref, w_ref, out0_ref, out1_ref):
    r = jnp.dot(x_ref[...], w_ref[...], preferred_element_type=jnp.float32)
    out0_ref[...] = r[:, :FH]
    out1_ref[...] = r[:, FH:]


def _mm1(x, w1):
    return pl.pallas_call(
        _mm1_body,
        grid=(N_ROW_BLKS,),
        in_specs=[
            pl.BlockSpec((ROW_BLK, NFEAT), lambda i: (i, 0)),
            pl.BlockSpec((NFEAT, NEMBED), lambda i: (0, 0)),
        ],
        out_specs=[pl.BlockSpec((ROW_BLK, FH), lambda i: (i, 0)),
                   pl.BlockSpec((ROW_BLK, FH), lambda i: (i, 0))],
        out_shape=[jax.ShapeDtypeStruct((N_NODES, FH), jnp.float32),
                   jax.ShapeDtypeStruct((N_NODES, FH), jnp.float32)],
    )(x, w1)


def _lsm_body(q0_ref, q1_ref, w2_ref, b2_ref, out_ref):
    s = jnp.concatenate([q0_ref[...], q1_ref[...]], axis=1)
    a = jnp.dot(s, w2_ref[...], preferred_element_type=jnp.float32) + b2_ref[0, :]
    m = jnp.max(a, axis=1, keepdims=True)
    e = jnp.exp(a - m)
    out_ref[...] = a - m - jnp.log(jnp.sum(e, axis=1, keepdims=True))


def _lsm(q0, q1, w2, b2):
    return pl.pallas_call(
        _lsm_body,
        grid=(N_ROW_BLKS,),
        in_specs=[
            pl.BlockSpec((ROW_BLK, FH), lambda i: (i, 0)),
            pl.BlockSpec((ROW_BLK, FH), lambda i: (i, 0)),
            pl.BlockSpec((NEMBED, NX), lambda i: (0, 0)),
            pl.BlockSpec((1, NX), lambda i: (0, 0)),
        ],
        out_specs=pl.BlockSpec((ROW_BLK, NX), lambda i: (i, 0)),
        out_shape=jax.ShapeDtypeStruct((N_NODES, NX), jnp.float32),
    )(q0, q1, w2, b2.reshape(1, NX))


def kernel(x, edge_index, W1, b1, W2, b2):
    src = edge_index[0].astype(jnp.int32)
    dst = edge_index[1].astype(jnp.int32)
    pad = E_PAD - N_EDGES
    src = jnp.concatenate([src, jnp.zeros((pad,), jnp.int32)])
    dst = jnp.concatenate([dst, jnp.full((pad,), N_NODES, jnp.int32)])
    src = jnp.concatenate(
        [src.reshape(TOTAL_CHUNKS, K), jnp.zeros((G, K), jnp.int32)])
    dst = jnp.concatenate(
        [dst.reshape(TOTAL_CHUNKS, K),
         jnp.full((G, K), N_NODES, jnp.int32)])
    zeros = jnp.zeros((NP, FH), jnp.float32)

    h0, h1 = _mm1(x, W1)
    q0, q1 = _gcn_sc(h0, h1, src, dst, zeros, b1.reshape(2, FH))
    return _lsm(q0, q1, W2, b2)
